# baseline (speedup 1.0000x reference)
import functools

import jax
import jax.numpy as jnp
from jax import lax
from jax.experimental import pallas as pl
from jax.experimental.pallas import tpu as pltpu
from jax.experimental.pallas import tpu_sc as plsc

_D = 128
_V = 10000
_T = 320000
_CH = 40
_LANES = 16
_W = _D // 2


def _encode(X, idx0, idx1):
    info = plsc.get_sparse_core_info()
    nw = info.num_cores * info.num_subcores
    per_w = _T // nw
    n_chunks = per_w // _CH

    mesh = plsc.VectorSubcoreMesh(core_axis_name="c", subcore_axis_name="s")

    @functools.partial(
        pl.kernel,
        mesh=mesh,
        out_type=jax.ShapeDtypeStruct((_T, _D), jnp.float32),
        scratch_types=[
            pltpu.VMEM_SHARED((_V, _D), jnp.float32),
            pltpu.VMEM((_CH,), jnp.int32),
            pltpu.VMEM((_CH,), jnp.int32),
            pltpu.VMEM((_CH, _D), jnp.float32),
            pltpu.VMEM((_CH, _D), jnp.float32),
            pltpu.SemaphoreType.DMA,
        ],
    )
    def k(x_hbm, i0_hbm, i1_hbm, out_hbm, xs, i0_v, i1_v, a_v, b_v, sem):
        wid = lax.axis_index("s") * info.num_cores + lax.axis_index("c")
        wbase = wid * per_w

        sid = lax.axis_index("s")
        stripe = pl.ds(sid * 624, 624)
        pltpu.sync_copy(x_hbm.at[stripe], xs.at[stripe])

        def tail_copy():
            tail = pl.ds(9984, 16)
            pltpu.sync_copy(x_hbm.at[tail], xs.at[tail])

        pl.when(sid == 15)(tail_copy)
        plsc.subcore_barrier()

        def chunk_body(g, carry):
            base = wbase + g * _CH
            pltpu.sync_copy(i0_hbm.at[pl.ds(base, _CH)], i0_v)
            pltpu.sync_copy(i1_hbm.at[pl.ds(base, _CH)], i1_v)
            cpa = pltpu.async_copy(xs.at[i0_v], a_v, sem)
            cpb = pltpu.async_copy(xs.at[i1_v], b_v, sem)
            cpa.wait()
            cpb.wait()

            def row_body(r, c2):
                for j in range(_D // _LANES):
                    sl = pl.ds(j * _LANES, _LANES)
                    a_v[r, sl] = a_v[r, sl] * b_v[r, sl]
                return c2

            lax.fori_loop(0, _CH, row_body, 0)
            pltpu.sync_copy(a_v, out_hbm.at[pl.ds(base, _CH)])
            return carry

        lax.fori_loop(0, n_chunks, chunk_body, 0)

    return k(X, idx0, idx1)


def _pack_table(X):
    Xb = X.astype(jnp.bfloat16)
    lo = jax.lax.bitcast_convert_type(Xb[:, :_W], jnp.uint16).astype(jnp.uint32)
    hi = jax.lax.bitcast_convert_type(Xb[:, _W:], jnp.uint16).astype(jnp.uint32)
    return jax.lax.bitcast_convert_type(lo | (hi << 16), jnp.int32)


def kernel(X, adj_t, tuples_coo):
    return _encode(X, tuples_coo[0], tuples_coo[1])


# speedup vs baseline: 2.4299x; 2.4299x over previous
import functools

import jax
import jax.numpy as jnp
from jax import lax
from jax.experimental import pallas as pl
from jax.experimental.pallas import tpu as pltpu
from jax.experimental.pallas import tpu_sc as plsc

_D = 128
_T = 320000
_CH = 80
_LANES = 16
_NBUF = 5
_LOOK = 3


def _encode(X, idx0, idx1):
    info = plsc.get_sparse_core_info()
    nw = info.num_cores * info.num_subcores
    per_w = _T // nw
    n_chunks = per_w // _CH

    mesh = plsc.VectorSubcoreMesh(core_axis_name="c", subcore_axis_name="s")

    scratch = [
        pltpu.VMEM((per_w,), jnp.int32),
        pltpu.VMEM((per_w,), jnp.int32),
    ]
    scratch += [pltpu.VMEM((_CH, _D), jnp.float32) for _ in range(2 * _NBUF)]
    scratch += [pltpu.SemaphoreType.DMA for _ in range(2 * _NBUF)]

    @functools.partial(
        pl.kernel,
        mesh=mesh,
        out_type=jax.ShapeDtypeStruct((_T, _D), jnp.float32),
        scratch_types=scratch,
    )
    def k(x_hbm, i0_hbm, i1_hbm, out_hbm, i0_v, i1_v, *bufs):
        a_s = bufs[0:_NBUF]
        b_s = bufs[_NBUF:2 * _NBUF]
        sg_s = bufs[2 * _NBUF:3 * _NBUF]
        st_s = bufs[3 * _NBUF:4 * _NBUF]

        wid = lax.axis_index("s") * info.num_cores + lax.axis_index("c")
        wbase = wid * per_w

        def issue_gather(g, s):
            sl = pl.ds(g * _CH, _CH)
            pltpu.async_copy(x_hbm.at[i0_v.at[sl]], a_s[s], sg_s[s])
            pltpu.async_copy(x_hbm.at[i1_v.at[sl]], b_s[s], sg_s[s])

        def wait_gather(s):
            sl = pl.ds(0, _CH)
            pltpu.make_async_copy(x_hbm.at[i0_v.at[sl]], a_s[s], sg_s[s]).wait()
            pltpu.make_async_copy(x_hbm.at[i1_v.at[sl]], b_s[s], sg_s[s]).wait()

        def drain_store(s):
            pltpu.make_async_copy(
                a_s[s], out_hbm.at[pl.ds(wbase, _CH)], st_s[s]).wait()

        def multiply(s):
            def row_body(r, c2):
                for j in range(_D // _LANES):
                    sl = pl.ds(j * _LANES, _LANES)
                    a_s[s][r, sl] = a_s[s][r, sl] * b_s[s][r, sl]
                return c2
            lax.fori_loop(0, _CH, row_body, 0)

        def issue_store(g, s):
            pltpu.async_copy(
                a_s[s], out_hbm.at[pl.ds(wbase + g * _CH, _CH)], st_s[s])

        pltpu.sync_copy(i0_hbm.at[pl.ds(wbase, per_w)], i0_v)
        pltpu.sync_copy(i1_hbm.at[pl.ds(wbase, per_w)], i1_v)

        for g0 in range(_LOOK):
            issue_gather(g0, g0)

        def block_body(p, carry):
            for b in range(_NBUF):
                g = _NBUF * p + b
                s_pre = (b + _LOOK) % _NBUF

                def prefetch():
                    pl.when(g >= _NBUF - _LOOK)(lambda: drain_store(s_pre))
                    issue_gather(g + _LOOK, s_pre)

                pl.when(g + _LOOK < n_chunks)(prefetch)
                wait_gather(b)
                multiply(b)
                issue_store(g, b)
            return carry

        lax.fori_loop(0, n_chunks // _NBUF, block_body, 0)

        for s in range(_NBUF):
            drain_store(s)

    return k(X, idx0, idx1)


def kernel(X, adj_t, tuples_coo):
    return _encode(X, tuples_coo[0], tuples_coo[1])
